# Initial kernel scaffold; baseline (speedup 1.0000x reference)
import jax
import jax.numpy as jnp
from jax.experimental import pallas as pl


def kernel(x, table):
    raise NotImplementedError("write your pallas kernel here")


# speedup vs baseline: 1.0942x; 1.0942x over previous
import functools

import jax
import jax.numpy as jnp
from jax import lax
from jax.experimental import pallas as pl
from jax.experimental.pallas import tpu as pltpu
from jax.experimental.pallas import tpu_sc as plsc

VOCAB = 1000000
EMBED_DIM = 32
NC = 2
NS = 16
NW = NC * NS

IDX_PER_STREAM = 128
CHUNK = 1024
NSUB = CHUNK // IDX_PER_STREAM


def _embed_body(n_chunks, x_hbm, table_hbm, out_hbm, idx_v, rows_v, sem):
    wid = lax.axis_index("s") * NC + lax.axis_index("c")
    base128 = wid * (n_chunks * NSUB)

    def chunk_body(i, carry):
        off128 = base128 + i * NSUB
        pltpu.sync_copy(x_hbm.at[pl.ds(off128, NSUB)], idx_v)
        cps = []
        for j in range(NSUB):
            cps.append(pltpu.async_copy(
                table_hbm.at[idx_v.at[j]],
                rows_v.at[pl.ds(j * IDX_PER_STREAM, IDX_PER_STREAM)],
                sem))
        for cp in cps:
            cp.wait()
        pltpu.sync_copy(rows_v, out_hbm.at[pl.ds(off128 * IDX_PER_STREAM, CHUNK)])
        return carry

    lax.fori_loop(0, n_chunks, chunk_body, 0)


def kernel(x, table):
    batch, hist = x.shape
    total = batch * hist
    assert total % (NW * CHUNK) == 0
    n_chunks = total // (NW * CHUNK)

    x2d = x.reshape(total // IDX_PER_STREAM, IDX_PER_STREAM)

    mesh = plsc.VectorSubcoreMesh(core_axis_name="c", subcore_axis_name="s",
                                  num_cores=NC, num_subcores=NS)
    out = pl.kernel(
        functools.partial(_embed_body, n_chunks),
        out_type=jax.ShapeDtypeStruct((total, EMBED_DIM), jnp.float32),
        mesh=mesh,
        scratch_types=[
            pltpu.VMEM((NSUB, IDX_PER_STREAM), jnp.int32),
            pltpu.VMEM((CHUNK, EMBED_DIM), jnp.float32),
            pltpu.SemaphoreType.DMA,
        ],
        compiler_params=pltpu.CompilerParams(use_tc_tiling_on_sc=False),
    )(x2d, table)
    return out.reshape(batch, hist, EMBED_DIM)


# speedup vs baseline: 1.1100x; 1.0145x over previous
import functools

import jax
import jax.numpy as jnp
from jax import lax
from jax.experimental import pallas as pl
from jax.experimental.pallas import tpu as pltpu
from jax.experimental.pallas import tpu_sc as plsc

VOCAB = 1000000
EMBED_DIM = 32
NC = 2
NS = 16
NW = NC * NS

IDX_PER_STREAM = 128
CHUNK = 1024
NSUB = CHUNK // IDX_PER_STREAM


def _embed_body(n_chunks, x_hbm, table_hbm, out_hbm, idx_v, rows_v,
                sem_idx, sem_gat, sem_out):
    wid = lax.axis_index("s") * NC + lax.axis_index("c")
    base128 = wid * (n_chunks * NSUB)

    def start_idx(i, slot):
        pltpu.async_copy(
            x_hbm.at[pl.ds(base128 + i * NSUB, NSUB)],
            idx_v.at[pl.ds(slot * NSUB, NSUB)],
            sem_idx)

    def wait_idx(slot):
        pltpu.make_async_copy(
            x_hbm.at[pl.ds(0, NSUB)],
            idx_v.at[pl.ds(slot * NSUB, NSUB)],
            sem_idx).wait()

    def start_out(i, slot):
        pltpu.async_copy(
            rows_v.at[pl.ds(slot * CHUNK, CHUNK)],
            out_hbm.at[pl.ds((base128 + i * NSUB) * IDX_PER_STREAM, CHUNK)],
            sem_out)

    def wait_out(slot):
        pltpu.make_async_copy(
            rows_v.at[pl.ds(slot * CHUNK, CHUNK)],
            out_hbm.at[pl.ds(0, CHUNK)],
            sem_out).wait()

    start_idx(0, 0)

    def chunk_body(i, carry):
        slot = lax.rem(i, 2)

        @pl.when(i + 1 < n_chunks)
        def _():
            start_idx(i + 1, 1 - slot)

        wait_idx(slot)

        @pl.when(i >= 2)
        def _():
            wait_out(slot)

        cps = []
        for j in range(NSUB):
            cps.append(pltpu.async_copy(
                table_hbm.at[idx_v.at[slot * NSUB + j]],
                rows_v.at[pl.ds(slot * CHUNK + j * IDX_PER_STREAM,
                                IDX_PER_STREAM)],
                sem_gat))
        for cp in cps:
            cp.wait()

        start_out(i, slot)
        return carry

    lax.fori_loop(0, n_chunks, chunk_body, 0)
    wait_out(0)
    wait_out(1)


def kernel(x, table):
    batch, hist = x.shape
    total = batch * hist
    assert total % (NW * CHUNK) == 0
    n_chunks = total // (NW * CHUNK)

    x2d = x.reshape(total // IDX_PER_STREAM, IDX_PER_STREAM)

    mesh = plsc.VectorSubcoreMesh(core_axis_name="c", subcore_axis_name="s",
                                  num_cores=NC, num_subcores=NS)
    out = pl.kernel(
        functools.partial(_embed_body, n_chunks),
        out_type=jax.ShapeDtypeStruct((total, EMBED_DIM), jnp.float32),
        mesh=mesh,
        scratch_types=[
            pltpu.VMEM((2 * NSUB, IDX_PER_STREAM), jnp.int32),
            pltpu.VMEM((2 * CHUNK, EMBED_DIM), jnp.float32),
            pltpu.SemaphoreType.DMA,
            pltpu.SemaphoreType.DMA,
            pltpu.SemaphoreType.DMA,
        ],
        compiler_params=pltpu.CompilerParams(use_tc_tiling_on_sc=False),
    )(x2d, table)
    return out.reshape(batch, hist, EMBED_DIM)


# speedup vs baseline: 1.4938x; 1.3457x over previous
import functools

import jax
import jax.numpy as jnp
from jax import lax
from jax.experimental import pallas as pl
from jax.experimental.pallas import tpu as pltpu
from jax.experimental.pallas import tpu_sc as plsc

VOCAB = 1000000
EMBED_DIM = 32
NC = 2
NS = 16
NW = NC * NS

LANE = 128
HIST = 50
BATCH_BLOCKS = 16384 // LANE
N_UNITS = HIST * BATCH_BLOCKS
UNITS_PER_W = N_UNITS // NW


def _embed_body(x_hbm, table_hbm, out_hbm, idx_v, buf_v, blk_v,
                sem_idx, sem_gat, sem_out):
    wid = lax.axis_index("s") * NC + lax.axis_index("c")
    u0 = wid * UNITS_PER_W

    pltpu.async_copy(x_hbm.at[pl.ds(u0, UNITS_PER_W)], idx_v, sem_idx).wait()

    rows16 = [lax.iota(jnp.int32, 16) + 16 * k for k in range(8)]

    def start_gather(j, slot):
        return pltpu.async_copy(
            table_hbm.at[idx_v.at[j]],
            buf_v.at[pl.ds(slot * LANE, LANE)],
            sem_gat)

    def start_out(j, slot):
        u = u0 + j
        h = u // BATCH_BLOCKS
        c = lax.rem(u, BATCH_BLOCKS)
        return pltpu.async_copy(
            blk_v.at[pl.ds(slot * EMBED_DIM, EMBED_DIM)],
            out_hbm.at[h, :, pl.ds(c * LANE, LANE)],
            sem_out)

    def wait_out(slot):
        pltpu.make_async_copy(
            blk_v.at[pl.ds(slot * EMBED_DIM, EMBED_DIM)],
            out_hbm.at[0, :, pl.ds(0, LANE)],
            sem_out).wait()

    def wait_gat(slot):
        pltpu.make_async_copy(
            table_hbm.at[idx_v.at[0]],
            buf_v.at[pl.ds(slot * LANE, LANE)],
            sem_gat).wait()

    def transpose_unit(slot):
        def d_body(d, carry):
            dvec = jnp.full((16,), 0, jnp.int32) + d
            for k in range(8):
                vals = plsc.load_gather(
                    buf_v, [slot * LANE + rows16[k], dvec])
                blk_v[slot * EMBED_DIM + d, pl.ds(16 * k, 16)] = vals
            return carry
        lax.fori_loop(0, EMBED_DIM, d_body, 0)

    start_gather(0, 0)

    def unit_body(j, carry):
        slot = lax.rem(j, 2)

        @pl.when(j + 1 < UNITS_PER_W)
        def _():
            start_gather(j + 1, 1 - slot)

        wait_gat(slot)

        @pl.when(j >= 2)
        def _():
            wait_out(slot)

        transpose_unit(slot)
        start_out(j, slot)
        return carry

    lax.fori_loop(0, UNITS_PER_W, unit_body, 0)
    wait_out(0)
    wait_out(1)


def kernel(x, table):
    batch, hist = x.shape
    assert hist == HIST and batch == BATCH_BLOCKS * LANE

    x2d = x.T.reshape(N_UNITS, LANE)

    mesh = plsc.VectorSubcoreMesh(core_axis_name="c", subcore_axis_name="s",
                                  num_cores=NC, num_subcores=NS)
    out = pl.kernel(
        _embed_body,
        out_type=jax.ShapeDtypeStruct((HIST, EMBED_DIM, batch), jnp.float32),
        mesh=mesh,
        scratch_types=[
            pltpu.VMEM((UNITS_PER_W, LANE), jnp.int32),
            pltpu.VMEM((2 * LANE, EMBED_DIM), jnp.float32),
            pltpu.VMEM((2 * EMBED_DIM, LANE), jnp.float32),
            pltpu.SemaphoreType.DMA,
            pltpu.SemaphoreType.DMA,
            pltpu.SemaphoreType.DMA,
        ],
        compiler_params=pltpu.CompilerParams(use_tc_tiling_on_sc=False,
                                             needs_layout_passes=False),
    )(x2d, table)
    return out.transpose(2, 0, 1)


# speedup vs baseline: 1.5013x; 1.0050x over previous
import jax
import jax.numpy as jnp
from jax import lax
from jax.experimental import pallas as pl
from jax.experimental.pallas import tpu as pltpu
from jax.experimental.pallas import tpu_sc as plsc

VOCAB = 1000000
EMBED_DIM = 32
NC = 2
NS = 16
NW = NC * NS

LANE = 128
HIST = 50
BATCH_BLOCKS = 16384 // LANE
N_UNITS = HIST * BATCH_BLOCKS
UNITS_PER_W = N_UNITS // NW


def _embed_body(xt_hbm, table_hbm, out_hbm, idx_v, buf_v, blk_v,
                sem_idx, sem_gat, sem_out):
    wid = lax.axis_index("s") * NC + lax.axis_index("c")
    u0 = wid * UNITS_PER_W

    rows16 = [lax.iota(jnp.int32, 16) + 16 * k for k in range(8)]
    dvecs = [jnp.full((16,), d, jnp.int32) for d in range(EMBED_DIM)]

    def start_idx(j, slot):
        u = u0 + j
        h = u // BATCH_BLOCKS
        c = lax.rem(u, BATCH_BLOCKS)
        pltpu.async_copy(
            xt_hbm.at[h, pl.ds(c * LANE, LANE)],
            idx_v.at[pl.ds(slot * LANE, LANE)],
            sem_idx)

    def wait_idx(slot):
        pltpu.make_async_copy(
            xt_hbm.at[0, pl.ds(0, LANE)],
            idx_v.at[pl.ds(slot * LANE, LANE)],
            sem_idx).wait()

    def start_gather(slot):
        pltpu.async_copy(
            table_hbm.at[idx_v.at[pl.ds(slot * LANE, LANE)]],
            buf_v.at[pl.ds(slot * LANE, LANE)],
            sem_gat)

    def wait_gat(slot):
        pltpu.make_async_copy(
            table_hbm.at[idx_v.at[pl.ds(0, LANE)]],
            buf_v.at[pl.ds(slot * LANE, LANE)],
            sem_gat).wait()

    def start_out(j, slot):
        u = u0 + j
        h = u // BATCH_BLOCKS
        c = lax.rem(u, BATCH_BLOCKS)
        pltpu.async_copy(
            blk_v.at[pl.ds(slot * EMBED_DIM, EMBED_DIM)],
            out_hbm.at[h, :, pl.ds(c * LANE, LANE)],
            sem_out)

    def wait_out(slot):
        pltpu.make_async_copy(
            blk_v.at[pl.ds(slot * EMBED_DIM, EMBED_DIM)],
            out_hbm.at[0, :, pl.ds(0, LANE)],
            sem_out).wait()

    def transpose_unit(slot):
        rows_s = [slot * LANE + r for r in rows16]
        blk_row0 = slot * EMBED_DIM
        for d in range(EMBED_DIM):
            for k in range(8):
                vals = plsc.load_gather(buf_v, [rows_s[k], dvecs[d]])
                blk_v[blk_row0 + d, pl.ds(16 * k, 16)] = vals

    start_idx(0, 0)
    start_idx(1, 1)
    wait_idx(0)
    start_gather(0)

    def unit_body(j, carry):
        slot = lax.rem(j, 2)

        @pl.when(j + 1 < UNITS_PER_W)
        def _():
            wait_idx(1 - slot)
            start_gather(1 - slot)

        wait_gat(slot)

        @pl.when(j + 2 < UNITS_PER_W)
        def _():
            start_idx(j + 2, slot)

        @pl.when(j >= 2)
        def _():
            wait_out(slot)

        transpose_unit(slot)
        start_out(j, slot)
        return carry

    lax.fori_loop(0, UNITS_PER_W, unit_body, 0)
    wait_out(0)
    wait_out(1)


def kernel(x, table):
    batch, hist = x.shape
    assert hist == HIST and batch == BATCH_BLOCKS * LANE

    mesh = plsc.VectorSubcoreMesh(core_axis_name="c", subcore_axis_name="s",
                                  num_cores=NC, num_subcores=NS)
    out = pl.kernel(
        _embed_body,
        out_type=jax.ShapeDtypeStruct((HIST, EMBED_DIM, batch), jnp.float32),
        mesh=mesh,
        scratch_types=[
            pltpu.VMEM((2 * LANE,), jnp.int32),
            pltpu.VMEM((2 * LANE, EMBED_DIM), jnp.float32),
            pltpu.VMEM((2 * EMBED_DIM, LANE), jnp.float32),
            pltpu.SemaphoreType.DMA,
            pltpu.SemaphoreType.DMA,
            pltpu.SemaphoreType.DMA,
        ],
        compiler_params=pltpu.CompilerParams(use_tc_tiling_on_sc=False,
                                             needs_layout_passes=False),
    )(x.T, table)
    return out.transpose(2, 0, 1)


# speedup vs baseline: 1.5042x; 1.0020x over previous
import jax
import jax.numpy as jnp
from jax import lax
from jax.experimental import pallas as pl
from jax.experimental.pallas import tpu as pltpu
from jax.experimental.pallas import tpu_sc as plsc

VOCAB = 1000000
EMBED_DIM = 32
NC = 2
NS = 16
NW = NC * NS

LANE = 128
HIST = 50
BATCH_BLOCKS = 16384 // LANE
N_UNITS = HIST * BATCH_BLOCKS
UNITS_PER_W = N_UNITS // NW

NBUF = 8


def _xprep_body(xt_ref, out_ref):
    out_ref[...] = xt_ref[...].reshape(N_UNITS, LANE)


def _xprep(xt):
    return pl.pallas_call(
        _xprep_body,
        out_shape=jax.ShapeDtypeStruct((N_UNITS, LANE), jnp.int32),
    )(xt)


def _embed_body(x2d_hbm, table_hbm, out_hbm, idx_v, buf_v, blk_v,
                sem_idx, sem_gat, sem_out):
    wid = lax.axis_index("s") * NC + lax.axis_index("c")
    u0 = wid * UNITS_PER_W

    rows16 = [lax.iota(jnp.int32, 16) + 16 * k for k in range(8)]
    dvecs = [jnp.full((16,), d, jnp.int32) for d in range(EMBED_DIM)]

    def start_idx(j):
        slot = lax.rem(j, NBUF)
        pltpu.async_copy(
            x2d_hbm.at[u0 + j],
            idx_v.at[pl.ds(slot * LANE, LANE)],
            sem_idx)

    def wait_idx(j):
        slot = lax.rem(j, NBUF)
        pltpu.make_async_copy(
            x2d_hbm.at[0],
            idx_v.at[pl.ds(slot * LANE, LANE)],
            sem_idx).wait()

    def start_gather(j):
        slot = lax.rem(j, NBUF)
        pltpu.async_copy(
            table_hbm.at[idx_v.at[pl.ds(slot * LANE, LANE)]],
            buf_v.at[pl.ds(slot * LANE, LANE)],
            sem_gat)

    def wait_gat(j):
        slot = lax.rem(j, NBUF)
        pltpu.make_async_copy(
            table_hbm.at[idx_v.at[pl.ds(0, LANE)]],
            buf_v.at[pl.ds(slot * LANE, LANE)],
            sem_gat).wait()

    def start_out(j):
        slot = lax.rem(j, 2)
        u = u0 + j
        h = u // BATCH_BLOCKS
        c = lax.rem(u, BATCH_BLOCKS)
        pltpu.async_copy(
            blk_v.at[pl.ds(slot * EMBED_DIM, EMBED_DIM)],
            out_hbm.at[h, :, pl.ds(c * LANE, LANE)],
            sem_out)

    def wait_out(j):
        slot = lax.rem(j, 2)
        pltpu.make_async_copy(
            blk_v.at[pl.ds(slot * EMBED_DIM, EMBED_DIM)],
            out_hbm.at[0, :, pl.ds(0, LANE)],
            sem_out).wait()

    def transpose_unit(j):
        gslot = lax.rem(j, NBUF)
        oslot = lax.rem(j, 2)
        rows_s = [gslot * LANE + r for r in rows16]
        blk_row0 = oslot * EMBED_DIM
        for d in range(EMBED_DIM):
            for k in range(8):
                vals = plsc.load_gather(buf_v, [rows_s[k], dvecs[d]])
                blk_v[blk_row0 + d, pl.ds(16 * k, 16)] = vals

    for j in range(NBUF):
        start_idx(j)
    for j in range(NBUF - 1):
        wait_idx(j)
        start_gather(j)

    def unit_body(j, carry):
        wait_gat(j)

        @pl.when(j + NBUF < UNITS_PER_W)
        def _():
            start_idx(j + NBUF)

        @pl.when(j + NBUF - 1 < UNITS_PER_W)
        def _():
            wait_idx(j + NBUF - 1)
            start_gather(j + NBUF - 1)

        @pl.when(j >= 2)
        def _():
            wait_out(j - 2)

        transpose_unit(j)
        start_out(j)
        return carry

    lax.fori_loop(0, UNITS_PER_W, unit_body, 0)
    wait_out(UNITS_PER_W - 2)
    wait_out(UNITS_PER_W - 1)


def kernel(x, table):
    batch, hist = x.shape
    assert hist == HIST and batch == BATCH_BLOCKS * LANE

    x2d = _xprep(x.T)

    mesh = plsc.VectorSubcoreMesh(core_axis_name="c", subcore_axis_name="s",
                                  num_cores=NC, num_subcores=NS)
    out = pl.kernel(
        _embed_body,
        out_type=jax.ShapeDtypeStruct((HIST, EMBED_DIM, batch), jnp.float32),
        mesh=mesh,
        scratch_types=[
            pltpu.VMEM((NBUF * LANE,), jnp.int32),
            pltpu.VMEM((NBUF * LANE, EMBED_DIM), jnp.float32),
            pltpu.VMEM((2 * EMBED_DIM, LANE), jnp.float32),
            pltpu.SemaphoreType.DMA,
            pltpu.SemaphoreType.DMA,
            pltpu.SemaphoreType.DMA,
        ],
        compiler_params=pltpu.CompilerParams(use_tc_tiling_on_sc=False,
                                             needs_layout_passes=False),
    )(x2d, table)
    return out.transpose(2, 0, 1)


# speedup vs baseline: 2.6301x; 1.7485x over previous
import functools

import jax
import jax.numpy as jnp
from jax import lax
from jax.experimental import pallas as pl
from jax.experimental.pallas import tpu as pltpu
from jax.experimental.pallas import tpu_sc as plsc

VOCAB = 1000000
EMBED_DIM = 32
NC = 2
NS = 16
NW = NC * NS

LANE = 128
HIST = 50
BATCH_BLOCKS = 16384 // LANE
N_UNITS = HIST * BATCH_BLOCKS

IDX_PER_STREAM = 128
CHUNK = 1024
NSUB = CHUNK // IDX_PER_STREAM


def _xprep_body(xt_ref, out_ref):
    y = xt_ref[...].reshape(N_UNITS, LANE).astype(jnp.float32)
    row = lax.broadcasted_iota(jnp.int32, (LANE, LANE), 0)
    col = lax.broadcasted_iota(jnp.int32, (LANE, LANE), 1)
    perm = (row == 32 * lax.rem(col, 4) + col // 4).astype(jnp.float32)
    z = jnp.dot(y, perm, preferred_element_type=jnp.float32,
                precision=lax.Precision.HIGHEST)
    out_ref[...] = z.astype(jnp.int32)


def _xprep(xt):
    return pl.pallas_call(
        _xprep_body,
        out_shape=jax.ShapeDtypeStruct((N_UNITS, LANE), jnp.int32),
    )(xt)


def _gather_body(n_chunks, x_hbm, table_hbm, out_hbm, idx_v, rows_v,
                 sem_idx, sem_gat, sem_out):
    wid = lax.axis_index("s") * NC + lax.axis_index("c")
    base128 = wid * (n_chunks * NSUB)

    def start_idx(i, slot):
        pltpu.async_copy(
            x_hbm.at[pl.ds(base128 + i * NSUB, NSUB)],
            idx_v.at[pl.ds(slot * NSUB, NSUB)],
            sem_idx)

    def wait_idx(slot):
        pltpu.make_async_copy(
            x_hbm.at[pl.ds(0, NSUB)],
            idx_v.at[pl.ds(slot * NSUB, NSUB)],
            sem_idx).wait()

    def start_out(i, slot):
        pltpu.async_copy(
            rows_v.at[pl.ds(slot * CHUNK, CHUNK)],
            out_hbm.at[pl.ds((base128 + i * NSUB) * IDX_PER_STREAM, CHUNK)],
            sem_out)

    def wait_out(slot):
        pltpu.make_async_copy(
            rows_v.at[pl.ds(slot * CHUNK, CHUNK)],
            out_hbm.at[pl.ds(0, CHUNK)],
            sem_out).wait()

    start_idx(0, 0)

    def chunk_body(i, carry):
        slot = lax.rem(i, 2)

        @pl.when(i + 1 < n_chunks)
        def _():
            start_idx(i + 1, 1 - slot)

        wait_idx(slot)

        @pl.when(i >= 2)
        def _():
            wait_out(slot)

        cps = []
        for j in range(NSUB):
            cps.append(pltpu.async_copy(
                table_hbm.at[idx_v.at[slot * NSUB + j]],
                rows_v.at[pl.ds(slot * CHUNK + j * IDX_PER_STREAM,
                                IDX_PER_STREAM)],
                sem_gat))
        for cp in cps:
            cp.wait()

        start_out(i, slot)
        return carry

    lax.fori_loop(0, n_chunks, chunk_body, 0)
    wait_out(0)
    wait_out(1)


def _gather(x2d, table):
    total = N_UNITS * LANE
    n_chunks = total // (NW * CHUNK)
    mesh = plsc.VectorSubcoreMesh(core_axis_name="c", subcore_axis_name="s",
                                  num_cores=NC, num_subcores=NS)
    return pl.kernel(
        functools.partial(_gather_body, n_chunks),
        out_type=jax.ShapeDtypeStruct((total, EMBED_DIM), jnp.float32),
        mesh=mesh,
        scratch_types=[
            pltpu.VMEM((2 * NSUB, IDX_PER_STREAM), jnp.int32),
            pltpu.VMEM((2 * CHUNK, EMBED_DIM), jnp.float32),
            pltpu.SemaphoreType.DMA,
            pltpu.SemaphoreType.DMA,
            pltpu.SemaphoreType.DMA,
        ],
        compiler_params=pltpu.CompilerParams(use_tc_tiling_on_sc=False),
    )(x2d, table)


def _transpose_body(in_ref, out_ref):
    x = in_ref[...]
    ri = lax.broadcasted_iota(jnp.int32, (LANE, LANE), 0)
    ci = lax.broadcasted_iota(jnp.int32, (LANE, LANE), 1)
    eye = (ri == ci).astype(jnp.float32)
    xT = lax.dot_general(eye, x, (((1,), (1,)), ((), ())),
                         preferred_element_type=jnp.float32,
                         precision=lax.Precision.HIGHEST)
    for k in range(4):
        for c in range(BATCH_BLOCKS):
            out_ref[0, :, c * LANE + 32 * k: c * LANE + 32 * k + 32] = (
                xT[32 * k: 32 * k + 32, c * 32: c * 32 + 32])


def _transpose(inter2):
    rows_per_h = BATCH_BLOCKS * LANE * EMBED_DIM // LANE
    return pl.pallas_call(
        _transpose_body,
        out_shape=jax.ShapeDtypeStruct((HIST, EMBED_DIM, BATCH_BLOCKS * LANE),
                                       jnp.float32),
        grid=(HIST,),
        in_specs=[pl.BlockSpec((rows_per_h, LANE), lambda h: (h, 0))],
        out_specs=pl.BlockSpec((1, EMBED_DIM, BATCH_BLOCKS * LANE),
                               lambda h: (h, 0, 0)),
    )(inter2)


def kernel(x, table):
    batch, hist = x.shape
    assert hist == HIST and batch == BATCH_BLOCKS * LANE

    x2d = _xprep(x.T)
    scout = _gather(x2d, table)
    inter2 = scout.reshape(N_UNITS * LANE * EMBED_DIM // LANE, LANE)
    out3 = _transpose(inter2)
    return out3.transpose(2, 0, 1)


# speedup vs baseline: 2.6406x; 1.0040x over previous
import functools

import jax
import jax.numpy as jnp
from jax import lax
from jax.experimental import pallas as pl
from jax.experimental.pallas import tpu as pltpu
from jax.experimental.pallas import tpu_sc as plsc

VOCAB = 1000000
EMBED_DIM = 32
NC = 2
NS = 16
NW = NC * NS

LANE = 128
HIST = 50
BATCH_BLOCKS = 16384 // LANE
N_UNITS = HIST * BATCH_BLOCKS

IDX_PER_STREAM = 128
CHUNK = 1024
NSUB = CHUNK // IDX_PER_STREAM


def _xprep_body(xt_ref, out_ref):
    y = xt_ref[...].reshape(N_UNITS, LANE).astype(jnp.float32)
    row = lax.broadcasted_iota(jnp.int32, (LANE, LANE), 0)
    col = lax.broadcasted_iota(jnp.int32, (LANE, LANE), 1)
    perm = (row == 32 * lax.rem(col, 4) + col // 4).astype(jnp.float32)
    z = jnp.dot(y, perm, preferred_element_type=jnp.float32,
                precision=lax.Precision.HIGHEST)
    out_ref[...] = z.astype(jnp.int32)


def _xprep(xt):
    return pl.pallas_call(
        _xprep_body,
        out_shape=jax.ShapeDtypeStruct((N_UNITS, LANE), jnp.int32),
    )(xt)


def _gather_body(n_chunks, x_hbm, table_hbm, out2d_hbm, idx_v, rows_v,
                 sem_idx, sem_gat, sem_out):
    wid = lax.axis_index("s") * NC + lax.axis_index("c")
    base128 = wid * (n_chunks * NSUB)

    def start_idx(i, slot):
        pltpu.async_copy(
            x_hbm.at[pl.ds(base128 + i * NSUB, NSUB)],
            idx_v.at[pl.ds(slot * NSUB, NSUB)],
            sem_idx)

    def wait_idx(slot):
        pltpu.make_async_copy(
            x_hbm.at[pl.ds(0, NSUB)],
            idx_v.at[pl.ds(slot * NSUB, NSUB)],
            sem_idx).wait()

    def start_out(i, slot):
        pltpu.async_copy(
            rows_v.at[pl.ds(slot * CHUNK, CHUNK)],
            out2d_hbm.at[pl.ds((base128 + i * NSUB) * IDX_PER_STREAM, CHUNK)],
            sem_out)

    def wait_out(slot):
        pltpu.make_async_copy(
            rows_v.at[pl.ds(slot * CHUNK, CHUNK)],
            out2d_hbm.at[pl.ds(0, CHUNK)],
            sem_out).wait()

    start_idx(0, 0)

    def chunk_body(i, carry):
        slot = lax.rem(i, 2)

        @pl.when(i + 1 < n_chunks)
        def _():
            start_idx(i + 1, 1 - slot)

        wait_idx(slot)

        @pl.when(i >= 2)
        def _():
            wait_out(slot)

        cps = []
        for j in range(NSUB):
            cps.append(pltpu.async_copy(
                table_hbm.at[idx_v.at[slot * NSUB + j]],
                rows_v.at[pl.ds(slot * CHUNK + j * IDX_PER_STREAM,
                                IDX_PER_STREAM)],
                sem_gat))
        for cp in cps:
            cp.wait()

        start_out(i, slot)
        return carry

    lax.fori_loop(0, n_chunks, chunk_body, 0)
    wait_out(0)
    wait_out(1)


def _gather(x2d, table):
    total = N_UNITS * LANE
    n_chunks = total // (NW * CHUNK)
    mesh = plsc.VectorSubcoreMesh(core_axis_name="c", subcore_axis_name="s",
                                  num_cores=NC, num_subcores=NS)
    return pl.kernel(
        functools.partial(_gather_body, n_chunks),
        out_type=jax.ShapeDtypeStruct((total, EMBED_DIM), jnp.float32),
        mesh=mesh,
        scratch_types=[
            pltpu.VMEM((2 * NSUB, IDX_PER_STREAM), jnp.int32),
            pltpu.VMEM((2 * CHUNK, EMBED_DIM), jnp.float32),
            pltpu.SemaphoreType.DMA,
            pltpu.SemaphoreType.DMA,
            pltpu.SemaphoreType.DMA,
        ],
        compiler_params=pltpu.CompilerParams(use_tc_tiling_on_sc=False),
    )(x2d, table)


def _transpose_body(in_ref, out_ref):
    x = in_ref[...].reshape(EMBED_DIM * BATCH_BLOCKS, LANE)
    ri = lax.broadcasted_iota(jnp.int32, (LANE, LANE), 0)
    ci = lax.broadcasted_iota(jnp.int32, (LANE, LANE), 1)
    eye = (ri == ci).astype(jnp.float32)
    xT = lax.dot_general(eye, x, (((1,), (1,)), ((), ())),
                         preferred_element_type=jnp.float32,
                         precision=lax.Precision.HIGHEST)
    for k in range(4):
        for c in range(BATCH_BLOCKS):
            out_ref[0, :, c * LANE + 32 * k: c * LANE + 32 * k + 32] = (
                xT[32 * k: 32 * k + 32, c * 32: c * 32 + 32])


def _transpose(flat):
    elems_per_h = BATCH_BLOCKS * LANE * EMBED_DIM
    return pl.pallas_call(
        _transpose_body,
        out_shape=jax.ShapeDtypeStruct((HIST, EMBED_DIM, BATCH_BLOCKS * LANE),
                                       jnp.float32),
        grid=(HIST,),
        in_specs=[pl.BlockSpec((elems_per_h,), lambda h: (h,))],
        out_specs=pl.BlockSpec((1, EMBED_DIM, BATCH_BLOCKS * LANE),
                               lambda h: (h, 0, 0)),
    )(flat)


def kernel(x, table):
    batch, hist = x.shape
    assert hist == HIST and batch == BATCH_BLOCKS * LANE

    x2d = _xprep(x.T)
    scout = _gather(x2d, table)
    out3 = _transpose(scout.reshape(-1))
    return out3.transpose(2, 0, 1)


# speedup vs baseline: 2.6432x; 1.0010x over previous
import functools

import jax
import jax.numpy as jnp
from jax import lax
from jax.experimental import pallas as pl
from jax.experimental.pallas import tpu as pltpu
from jax.experimental.pallas import tpu_sc as plsc

VOCAB = 1000000
EMBED_DIM = 32
NC = 2
NS = 16
NW = NC * NS

LANE = 128
HIST = 50
BATCH_BLOCKS = 16384 // LANE
N_UNITS = HIST * BATCH_BLOCKS

IDX_PER_STREAM = 128
CHUNK = 1024
NSUB = CHUNK // IDX_PER_STREAM


def _xprep_body(xt_ref, out_ref):
    y = xt_ref[...].reshape(N_UNITS, LANE).astype(jnp.float32)
    row = lax.broadcasted_iota(jnp.int32, (LANE, LANE), 0)
    col = lax.broadcasted_iota(jnp.int32, (LANE, LANE), 1)
    perm = (row == 32 * lax.rem(col, 4) + col // 4).astype(jnp.float32)
    z = jnp.dot(y, perm, preferred_element_type=jnp.float32,
                precision=lax.Precision.HIGHEST)
    out_ref[...] = z.astype(jnp.int32).reshape(N_UNITS * LANE)


def _xprep(xt):
    return pl.pallas_call(
        _xprep_body,
        out_shape=jax.ShapeDtypeStruct((N_UNITS * LANE,), jnp.int32),
    )(xt)


def _gather_body(n_chunks, x_hbm, table_hbm, out2d_hbm, idx_v, rows_v,
                 sem_idx, sem_gat, sem_out):
    wid = lax.axis_index("s") * NC + lax.axis_index("c")
    base128 = wid * (n_chunks * NSUB)

    def start_idx(i, slot):
        pltpu.async_copy(
            x_hbm.at[pl.ds((base128 + i * NSUB) * IDX_PER_STREAM, CHUNK)],
            idx_v.at[pl.ds(slot * CHUNK, CHUNK)],
            sem_idx)

    def wait_idx(slot):
        pltpu.make_async_copy(
            x_hbm.at[pl.ds(0, CHUNK)],
            idx_v.at[pl.ds(slot * CHUNK, CHUNK)],
            sem_idx).wait()

    def start_out(i, slot):
        pltpu.async_copy(
            rows_v.at[pl.ds(slot * CHUNK, CHUNK)],
            out2d_hbm.at[pl.ds((base128 + i * NSUB) * IDX_PER_STREAM, CHUNK)],
            sem_out)

    def wait_out(slot):
        pltpu.make_async_copy(
            rows_v.at[pl.ds(slot * CHUNK, CHUNK)],
            out2d_hbm.at[pl.ds(0, CHUNK)],
            sem_out).wait()

    start_idx(0, 0)

    def chunk_body(i, carry):
        slot = lax.rem(i, 2)

        @pl.when(i + 1 < n_chunks)
        def _():
            start_idx(i + 1, 1 - slot)

        wait_idx(slot)

        @pl.when(i >= 2)
        def _():
            wait_out(slot)

        cps = []
        for j in range(NSUB):
            cps.append(pltpu.async_copy(
                table_hbm.at[idx_v.at[pl.ds(slot * CHUNK
                                            + j * IDX_PER_STREAM,
                                            IDX_PER_STREAM)]],
                rows_v.at[pl.ds(slot * CHUNK + j * IDX_PER_STREAM,
                                IDX_PER_STREAM)],
                sem_gat))
        for cp in cps:
            cp.wait()

        start_out(i, slot)
        return carry

    lax.fori_loop(0, n_chunks, chunk_body, 0)
    wait_out(0)
    wait_out(1)


def _gather(x2d, table):
    total = N_UNITS * LANE
    n_chunks = total // (NW * CHUNK)
    mesh = plsc.VectorSubcoreMesh(core_axis_name="c", subcore_axis_name="s",
                                  num_cores=NC, num_subcores=NS)
    return pl.kernel(
        functools.partial(_gather_body, n_chunks),
        out_type=jax.ShapeDtypeStruct((total, EMBED_DIM), jnp.float32),
        mesh=mesh,
        scratch_types=[
            pltpu.VMEM((2 * CHUNK,), jnp.int32),
            pltpu.VMEM((2 * CHUNK, EMBED_DIM), jnp.float32),
            pltpu.SemaphoreType.DMA,
            pltpu.SemaphoreType.DMA,
            pltpu.SemaphoreType.DMA,
        ],
        compiler_params=pltpu.CompilerParams(use_tc_tiling_on_sc=False),
    )(x2d, table)


def _transpose_body(in_ref, out_ref):
    x = in_ref[...].reshape(EMBED_DIM * BATCH_BLOCKS, LANE)
    ri = lax.broadcasted_iota(jnp.int32, (LANE, LANE), 0)
    ci = lax.broadcasted_iota(jnp.int32, (LANE, LANE), 1)
    eye = (ri == ci).astype(jnp.float32)
    xT = lax.dot_general(eye, x, (((1,), (1,)), ((), ())),
                         preferred_element_type=jnp.float32,
                         precision=lax.Precision.HIGHEST)
    for k in range(4):
        for c in range(BATCH_BLOCKS):
            out_ref[0, :, c * LANE + 32 * k: c * LANE + 32 * k + 32] = (
                xT[32 * k: 32 * k + 32, c * 32: c * 32 + 32])


def _transpose(flat):
    elems_per_h = BATCH_BLOCKS * LANE * EMBED_DIM
    return pl.pallas_call(
        _transpose_body,
        out_shape=jax.ShapeDtypeStruct((HIST, EMBED_DIM, BATCH_BLOCKS * LANE),
                                       jnp.float32),
        grid=(HIST,),
        in_specs=[pl.BlockSpec((elems_per_h,), lambda h: (h,))],
        out_specs=pl.BlockSpec((1, EMBED_DIM, BATCH_BLOCKS * LANE),
                               lambda h: (h, 0, 0)),
    )(flat)


def kernel(x, table):
    batch, hist = x.shape
    assert hist == HIST and batch == BATCH_BLOCKS * LANE

    x2d = _xprep(x.T)
    scout = _gather(x2d, table)
    out3 = _transpose(scout.reshape(-1))
    return out3.transpose(2, 0, 1)
